# Initial kernel scaffold; baseline (speedup 1.0000x reference)
import jax
import jax.numpy as jnp
from jax.experimental import pallas as pl


def kernel(emb, conv1_w, conv1_b, conv2_w, conv2_b, conv3_w, conv3_b, fc1_wq, fc1_s, fc1_b, fc2_wq, fc2_s, fc2_b, fc3_w, fc3_b, token_ids):
    raise NotImplementedError("write your pallas kernel here")


# speedup vs baseline: 1.0325x; 1.0325x over previous
import functools

import jax
import jax.numpy as jnp
from jax.experimental import pallas as pl
from jax.experimental.pallas import tpu as pltpu

_VMEM_LIMIT = 32 * 1024 * 1024


def _ceil8(x):
    return ((x + 7) // 8) * 8


def _s1_body(ia_ref, ib_ref, ic_ref, ga_ref, gb_ref, bt_ref, o_ref, *, vocab):
    hp = ia_ref.shape[1]

    def oh2(r1_ref, r2_ref):
        i1, i2 = r1_ref[0], r2_ref[0]
        iota = jax.lax.broadcasted_iota(jnp.int32, (hp, 2 * vocab), 1)
        return ((iota == i1) | (iota == i2 + vocab)).astype(ga_ref.dtype)

    oab, obc = oh2(ia_ref, ib_ref), oh2(ib_ref, ic_ref)
    ga, gb = ga_ref[...], gb_ref[...]
    v00 = jnp.dot(oab, ga, preferred_element_type=jnp.float32)
    v10 = jnp.dot(obc, ga, preferred_element_type=jnp.float32)
    v01 = jnp.dot(oab, gb, preferred_element_type=jnp.float32)
    v11 = jnp.dot(obc, gb, preferred_element_type=jnp.float32)
    p = jnp.maximum(jnp.maximum(v00, v01), jnp.maximum(v10, v11))
    o_ref[0] = jnp.maximum(p + bt_ref[...], 0.0).astype(o_ref.dtype)


def _embed_conv1(token_ids, table, w1, b1, cdt):
    B, S = token_ids.shape
    V, E = table.shape
    C1 = w1.shape[-1]
    Hp, Wp = (S - 1) // 2, (E - 1) // 2

    even = 2 * jnp.arange(Wp)
    col = [jnp.take(table, even + d, axis=1) for d in range(3)]

    def filt(dy, dj):
        g = (col[dj][:, :, None] * w1[dy, 0, :]
             + col[dj + 1][:, :, None] * w1[dy, 1, :])
        return g.reshape(V, Wp * C1)

    ga = jnp.concatenate([filt(0, 0), filt(1, 0)], axis=0).astype(cdt)
    gb = jnp.concatenate([filt(0, 1), filt(1, 1)], axis=0).astype(cdt)
    bt = jnp.tile(b1.astype(jnp.float32), Wp).reshape(1, Wp * C1)

    ids = token_ids.astype(jnp.int32)
    ia = ids[:, 0:2 * Hp:2].reshape(B, Hp, 1)
    ib = ids[:, 1:2 * Hp + 1:2].reshape(B, Hp, 1)
    ic = ids[:, 2:2 * Hp + 2:2].reshape(B, Hp, 1)

    return pl.pallas_call(
        functools.partial(_s1_body, vocab=V),
        out_shape=jax.ShapeDtypeStruct((B, Hp, Wp * C1), cdt),
        grid=(B,),
        in_specs=[
            pl.BlockSpec((1, Hp, 1), lambda b: (b, 0, 0)),
            pl.BlockSpec((1, Hp, 1), lambda b: (b, 0, 0)),
            pl.BlockSpec((1, Hp, 1), lambda b: (b, 0, 0)),
            pl.BlockSpec((2 * V, Wp * C1), lambda b: (0, 0)),
            pl.BlockSpec((2 * V, Wp * C1), lambda b: (0, 0)),
            pl.BlockSpec((1, Wp * C1), lambda b: (0, 0)),
        ],
        out_specs=pl.BlockSpec((1, Hp, Wp * C1), lambda b: (b, 0, 0)),
        compiler_params=pltpu.CompilerParams(
            dimension_semantics=("parallel",),
            vmem_limit_bytes=_VMEM_LIMIT),
    )(ia, ib, ic, ga, gb, bt)


def _conv_body(x_ref, w_ref, b_ref, o_ref, p_ref, d_ref, cv_ref, sh_ref, *,
               W, kh, kw, cin, cout, n_conv, n_rp, n_sel, n_shift, hp, wp):
    for dx in range(kw):
        p_ref[pl.ds(0, n_rp), pl.ds(dx * cin, cin)] = x_ref[0, pl.ds(dx, n_rp), :]
    d_ref[...] = jnp.dot(p_ref[...], w_ref[...],
                         preferred_element_type=jnp.float32)
    acc = d_ref[pl.ds(0, n_conv), pl.ds(0, cout)]
    for dy in range(1, kh):
        acc = acc + d_ref[pl.ds(dy * W, n_conv), pl.ds(dy * cout, cout)]
    cv_ref[pl.ds(0, n_conv), :] = jnp.maximum(acc + b_ref[...], 0.0)

    sh_ref[pl.ds(0, n_shift), :] = jnp.maximum(
        cv_ref[pl.ds(0, n_shift), :], cv_ref[pl.ds(1, n_shift), :])
    q = jnp.maximum(sh_ref[pl.ds(0, n_sel), :], sh_ref[pl.ds(W, n_sel), :])
    cv_ref[pl.ds(0, n_sel), :] = q
    o_ref[...] = jnp.zeros_like(o_ref)
    for ip in range(hp):
        o_ref[0, pl.ds(ip * wp, wp), :] = cv_ref[
            pl.ds(2 * ip * W, wp, 2), :].astype(o_ref.dtype)


def _conv_pool(x, w, b, *, H, W, kh, kw, out_dtype):
    B, rows_pad, cin = x.shape
    cout = w.shape[-1] // kh
    ho, wo = H - kh + 1, W - kw + 1
    hp, wp = ho // 2, wo // 2
    n_conv = (ho - 1) * W + wo
    n_rp = n_conv + (kh - 1) * W
    n_sel = 2 * (hp - 1) * W + 2 * (wp - 1) + 1
    n_shift = n_sel + W
    out_rows = _ceil8(hp * wp)

    body = functools.partial(
        _conv_body, W=W, kh=kh, kw=kw, cin=cin, cout=cout, n_conv=n_conv,
        n_rp=n_rp, n_sel=n_sel, n_shift=n_shift, hp=hp, wp=wp)
    return pl.pallas_call(
        body,
        out_shape=jax.ShapeDtypeStruct((B, out_rows, cout), out_dtype),
        grid=(B,),
        in_specs=[
            pl.BlockSpec((1, rows_pad, cin), lambda b: (b, 0, 0)),
            pl.BlockSpec((kw * cin, kh * cout), lambda b: (0, 0)),
            pl.BlockSpec((1, cout), lambda b: (0, 0)),
        ],
        out_specs=pl.BlockSpec((1, out_rows, cout), lambda b: (b, 0, 0)),
        scratch_shapes=[
            pltpu.VMEM((_ceil8(n_rp), kw * cin), x.dtype),
            pltpu.VMEM((_ceil8(n_rp), kh * cout), jnp.float32),
            pltpu.VMEM((_ceil8(n_conv), cout), jnp.float32),
            pltpu.VMEM((_ceil8(n_shift), cout), jnp.float32),
        ],
        compiler_params=pltpu.CompilerParams(
            dimension_semantics=("parallel",),
            vmem_limit_bytes=_VMEM_LIMIT),
    )(x, w, b.reshape(1, cout).astype(jnp.float32))


def _fc1_body(x_ref, w_ref, s_ref, b_ref, o_ref):
    w = w_ref[...].astype(x_ref.dtype)
    acc = jnp.dot(x_ref[...], w, preferred_element_type=jnp.float32)
    o_ref[...] = jnp.maximum(acc * s_ref[...] + b_ref[...], 0.0
                             ).astype(o_ref.dtype)


def _fc23_body(x_ref, w2_ref, s2_ref, b2_ref, w3_ref, b3_ref, o_ref, h_ref):
    w2 = w2_ref[...].astype(x_ref.dtype)
    a = jnp.dot(x_ref[...], w2, preferred_element_type=jnp.float32)
    h_ref[...] = jnp.maximum(a * s2_ref[...] + b2_ref[...], 0.0
                             ).astype(h_ref.dtype)
    o_ref[...] = (jnp.dot(h_ref[...], w3_ref[...],
                          preferred_element_type=jnp.float32) + b3_ref[...])


def _fc1(x, wq, s, b, cdt, tm):
    M, K = x.shape
    N = wq.shape[1]
    tn = 128
    out = pl.pallas_call(
        _fc1_body,
        out_shape=jax.ShapeDtypeStruct((M, N), cdt),
        grid=(M // tm, N // tn),
        in_specs=[
            pl.BlockSpec((tm, K), lambda i, j: (i, 0)),
            pl.BlockSpec((K, tn), lambda i, j: (0, j)),
            pl.BlockSpec((1, tn), lambda i, j: (0, j)),
            pl.BlockSpec((1, tn), lambda i, j: (0, j)),
        ],
        out_specs=pl.BlockSpec((tm, tn), lambda i, j: (i, j)),
        compiler_params=pltpu.CompilerParams(
            dimension_semantics=("parallel", "parallel"),
            vmem_limit_bytes=_VMEM_LIMIT),
    )(x, wq, s.reshape(1, N).astype(jnp.float32),
      b.reshape(1, N).astype(jnp.float32))
    return out


def _fc23(x, w2q, s2, b2, w3, b3, cdt, tm):
    M, K = x.shape
    N2 = w2q.shape[1]
    N3 = w3.shape[1]
    return pl.pallas_call(
        _fc23_body,
        out_shape=jax.ShapeDtypeStruct((M, N3), jnp.float32),
        grid=(M // tm,),
        in_specs=[
            pl.BlockSpec((tm, K), lambda i: (i, 0)),
            pl.BlockSpec((K, N2), lambda i: (0, 0)),
            pl.BlockSpec((1, N2), lambda i: (0, 0)),
            pl.BlockSpec((1, N2), lambda i: (0, 0)),
            pl.BlockSpec((N2, N3), lambda i: (0, 0)),
            pl.BlockSpec((1, N3), lambda i: (0, 0)),
        ],
        out_specs=pl.BlockSpec((tm, N3), lambda i: (i, 0)),
        scratch_shapes=[pltpu.VMEM((tm, N2), cdt)],
        compiler_params=pltpu.CompilerParams(
            dimension_semantics=("parallel",),
            vmem_limit_bytes=_VMEM_LIMIT),
    )(x, w2q, s2.reshape(1, N2).astype(jnp.float32),
      b2.reshape(1, N2).astype(jnp.float32),
      w3.astype(cdt), b3.reshape(1, N3).astype(jnp.float32))


def kernel(emb, conv1_w, conv1_b, conv2_w, conv2_b, conv3_w, conv3_b,
           fc1_wq, fc1_s, fc1_b, fc2_wq, fc2_s, fc2_b, fc3_w, fc3_b,
           token_ids):
    cdt = jnp.float32
    B = token_ids.shape[0]
    C1 = conv1_b.shape[0]

    x = _embed_conv1(token_ids, emb, conv1_w, conv1_b, cdt)
    hp1 = x.shape[1]
    x = x.reshape(B, hp1 * hp1, C1)
    x = jnp.pad(x, ((0, 0), (0, _ceil8(hp1 * hp1) - hp1 * hp1), (0, 0)))

    w2 = conv2_w.reshape(3, 3, 16, 32).transpose(1, 2, 0, 3).reshape(48, 96)
    w3 = conv3_w.reshape(4, 4, 32, 64).transpose(1, 2, 0, 3).reshape(128, 256)
    x = _conv_pool(x, w2.astype(cdt), conv2_b, H=49, W=49, kh=3, kw=3,
                   out_dtype=cdt)
    x = _conv_pool(x, w3.astype(cdt), conv3_b, H=23, W=23, kh=4, kw=4,
                   out_dtype=cdt)

    x = x[:, :100, :].reshape(B, 6400)
    x = _fc1(x, fc1_wq, fc1_s, fc1_b, cdt, tm=128)
    return _fc23(x, fc2_wq, fc2_s, fc2_b, fc3_w, fc3_b, cdt, tm=128)


# speedup vs baseline: 4.0589x; 3.9311x over previous
import functools

import jax
import jax.numpy as jnp
from jax.experimental import pallas as pl
from jax.experimental.pallas import tpu as pltpu

_VMEM_LIMIT = 32 * 1024 * 1024
_NB = 8


def _ceil8(x):
    return ((x + 7) // 8) * 8


def _s1_body(ia_ref, ib_ref, ic_ref, ga_ref, gb_ref, bt_ref, o_ref, *,
             vocab, nb, hp):
    m = nb * hp
    iota = jax.lax.broadcasted_iota(jnp.int32, (m, 2 * vocab), 1)
    ia = ia_ref[...].reshape(m, 1)
    ib = ib_ref[...].reshape(m, 1)
    ic = ic_ref[...].reshape(m, 1)
    f = ga_ref.dtype
    oab = ((iota == ia) | (iota == ib + vocab)).astype(f)
    obc = ((iota == ib) | (iota == ic + vocab)).astype(f)
    ga, gb = ga_ref[...], gb_ref[...]
    o_ref[...] = jnp.dot(oab, ga, preferred_element_type=jnp.float32
                         ).astype(o_ref.dtype)
    for oh, g in ((obc, ga), (oab, gb), (obc, gb)):
        o_ref[...] = jnp.maximum(
            o_ref[...],
            jnp.dot(oh, g, preferred_element_type=jnp.float32
                    ).astype(o_ref.dtype))
    o_ref[...] = jnp.maximum(o_ref[...] + bt_ref[...].astype(o_ref.dtype), 0.0)


def _embed_conv1(token_ids, table, w1, b1, cdt):
    B, S = token_ids.shape
    V, E = table.shape
    C1 = w1.shape[-1]
    Hp, Wp = (S - 1) // 2, (E - 1) // 2

    even = 2 * jnp.arange(Wp)
    col = [jnp.take(table, even + d, axis=1) for d in range(3)]

    def filt(dy, dj):
        g = (col[dj][:, :, None] * w1[dy, 0, :]
             + col[dj + 1][:, :, None] * w1[dy, 1, :])
        return g.reshape(V, Wp * C1)

    ga = jnp.concatenate([filt(0, 0), filt(1, 0)], axis=0).astype(cdt)
    gb = jnp.concatenate([filt(0, 1), filt(1, 1)], axis=0).astype(cdt)
    bt = jnp.tile(b1.astype(jnp.float32), Wp).reshape(1, Wp * C1)

    ids = token_ids.astype(jnp.int32)
    ia = ids[:, 0:2 * Hp:2].reshape(B, Hp, 1)
    ib = ids[:, 1:2 * Hp + 1:2].reshape(B, Hp, 1)
    ic = ids[:, 2:2 * Hp + 2:2].reshape(B, Hp, 1)

    body = functools.partial(_s1_body, vocab=V, nb=_NB, hp=Hp)
    return pl.pallas_call(
        body,
        out_shape=jax.ShapeDtypeStruct((B * Hp, Wp * C1), cdt),
        grid=(B // _NB,),
        in_specs=[
            pl.BlockSpec((_NB, Hp, 1), lambda b: (b, 0, 0)),
            pl.BlockSpec((_NB, Hp, 1), lambda b: (b, 0, 0)),
            pl.BlockSpec((_NB, Hp, 1), lambda b: (b, 0, 0)),
            pl.BlockSpec((2 * V, Wp * C1), lambda b: (0, 0)),
            pl.BlockSpec((2 * V, Wp * C1), lambda b: (0, 0)),
            pl.BlockSpec((1, Wp * C1), lambda b: (0, 0)),
        ],
        out_specs=pl.BlockSpec((_NB * Hp, Wp * C1), lambda b: (b, 0)),
        compiler_params=pltpu.CompilerParams(
            dimension_semantics=("parallel",),
            vmem_limit_bytes=_VMEM_LIMIT),
    )(ia, ib, ic, ga, gb, bt)


def _banded(wmat, kh, kw, cin, cout, win, wo, dtype):
    w4 = wmat.reshape(kh, kw, cin, cout).astype(jnp.float32)
    x = jnp.arange(wo)
    order = jnp.concatenate([x[0::2], x[1::2]])
    j = jnp.arange(win)
    a = jnp.zeros((kh, win, cin, wo, cout), jnp.float32)
    for dx in range(kw):
        e = (j[:, None] == order[None, :] + dx).astype(jnp.float32)
        a = a + jnp.einsum('jx,kic->kjixc', e, w4[:, dx])
    return a.reshape(kh, win * cin, wo * cout).astype(dtype)


def _conv_body(x_ref, a_ref, bt_ref, o_ref, d_ref, hs_ref, hm_ref, *,
               nb, kh, pin, pout, ho, hp, wp, cout, half):
    mv = nb * pin - (kh - 1)
    d_ref[pl.ds(0, mv), :] = jnp.dot(
        x_ref[pl.ds(0, mv), :], a_ref[0],
        preferred_element_type=jnp.float32)
    for dy in range(1, kh):
        d_ref[pl.ds(0, mv), :] = d_ref[pl.ds(0, mv), :] + jnp.dot(
            x_ref[pl.ds(dy, mv), :], a_ref[dy],
            preferred_element_type=jnp.float32)
    nw = wp * cout
    hs_ref[pl.ds(0, mv), :] = jnp.maximum(
        d_ref[pl.ds(0, mv), pl.ds(0, nw)],
        d_ref[pl.ds(0, mv), pl.ds(half, nw)])
    chunks = (nw + 127) // 128
    for c in range(chunks):
        wc = min(128, nw - c * 128)
        hm_ref[c, pl.ds(0, mv - 1), pl.ds(0, wc)] = jnp.maximum(
            hs_ref[pl.ds(0, mv - 1), pl.ds(c * 128, wc)],
            hs_ref[pl.ds(1, mv - 1), pl.ds(c * 128, wc)])
    o_ref[...] = jnp.zeros_like(o_ref)
    for i in range(nb):
        for c in range(chunks):
            wc = min(128, nw - c * 128)
            o_ref[pl.ds(i * pout, hp), pl.ds(c * 128, wc)] = jnp.maximum(
                hm_ref[c, pl.ds(i * pin, hp, 2), pl.ds(0, wc)]
                + bt_ref[0, pl.ds(c * 128, wc)], 0.0).astype(o_ref.dtype)


def _conv_lane(x, a, b, *, kh, pin, ho, hp, wp, cout, wo, out_dtype):
    rows, K = x.shape
    B = rows // pin
    N = a.shape[-1]
    pout = _ceil8(hp)
    half = ((wo + 1) // 2) * cout
    bt = jnp.tile(b.astype(jnp.float32), wp).reshape(1, wp * cout)
    mv_pad = _ceil8(_NB * pin)

    body = functools.partial(
        _conv_body, nb=_NB, kh=kh, pin=pin, pout=pout, ho=ho, hp=hp, wp=wp,
        cout=cout, half=half)
    return pl.pallas_call(
        body,
        out_shape=jax.ShapeDtypeStruct((B * pout, wp * cout), out_dtype),
        grid=(B // _NB,),
        in_specs=[
            pl.BlockSpec((_NB * pin, K), lambda bb: (bb, 0)),
            pl.BlockSpec((kh, K, N), lambda bb: (0, 0, 0)),
            pl.BlockSpec((1, wp * cout), lambda bb: (0, 0)),
        ],
        out_specs=pl.BlockSpec((_NB * pout, wp * cout), lambda bb: (bb, 0)),
        scratch_shapes=[
            pltpu.VMEM((mv_pad, N), jnp.float32),
            pltpu.VMEM((mv_pad, wp * cout), jnp.float32),
            pltpu.VMEM(((wp * cout + 127) // 128, mv_pad, 128), jnp.float32),
        ],
        compiler_params=pltpu.CompilerParams(
            dimension_semantics=("parallel",),
            vmem_limit_bytes=_VMEM_LIMIT),
    )(x, a, bt)


def _fc1_body(x_ref, w_ref, s_ref, b_ref, o_ref):
    w = w_ref[...].astype(x_ref.dtype)
    acc = jnp.dot(x_ref[...], w, preferred_element_type=jnp.float32)
    o_ref[...] = jnp.maximum(acc * s_ref[...] + b_ref[...], 0.0
                             ).astype(o_ref.dtype)


def _fc23_body(x_ref, w2_ref, s2_ref, b2_ref, w3_ref, b3_ref, o_ref, h_ref):
    w2 = w2_ref[...].astype(x_ref.dtype)
    a = jnp.dot(x_ref[...], w2, preferred_element_type=jnp.float32)
    h_ref[...] = jnp.maximum(a * s2_ref[...] + b2_ref[...], 0.0
                             ).astype(h_ref.dtype)
    o_ref[...] = (jnp.dot(h_ref[...], w3_ref[...],
                          preferred_element_type=jnp.float32) + b3_ref[...])


def _fc1(x, wq, s, b, cdt, tm):
    M, K = x.shape
    N = wq.shape[1]
    tn = 128
    return pl.pallas_call(
        _fc1_body,
        out_shape=jax.ShapeDtypeStruct((M, N), cdt),
        grid=(M // tm, N // tn),
        in_specs=[
            pl.BlockSpec((tm, K), lambda i, j: (i, 0)),
            pl.BlockSpec((K, tn), lambda i, j: (0, j)),
            pl.BlockSpec((1, tn), lambda i, j: (0, j)),
            pl.BlockSpec((1, tn), lambda i, j: (0, j)),
        ],
        out_specs=pl.BlockSpec((tm, tn), lambda i, j: (i, j)),
        compiler_params=pltpu.CompilerParams(
            dimension_semantics=("parallel", "parallel"),
            vmem_limit_bytes=_VMEM_LIMIT),
    )(x, wq, s.reshape(1, N).astype(jnp.float32),
      b.reshape(1, N).astype(jnp.float32))


def _fc23(x, w2q, s2, b2, w3, b3, cdt, tm):
    M, K = x.shape
    N2 = w2q.shape[1]
    N3 = w3.shape[1]
    return pl.pallas_call(
        _fc23_body,
        out_shape=jax.ShapeDtypeStruct((M, N3), jnp.float32),
        grid=(M // tm,),
        in_specs=[
            pl.BlockSpec((tm, K), lambda i: (i, 0)),
            pl.BlockSpec((K, N2), lambda i: (0, 0)),
            pl.BlockSpec((1, N2), lambda i: (0, 0)),
            pl.BlockSpec((1, N2), lambda i: (0, 0)),
            pl.BlockSpec((N2, N3), lambda i: (0, 0)),
            pl.BlockSpec((1, N3), lambda i: (0, 0)),
        ],
        out_specs=pl.BlockSpec((tm, N3), lambda i: (i, 0)),
        scratch_shapes=[pltpu.VMEM((tm, N2), cdt)],
        compiler_params=pltpu.CompilerParams(
            dimension_semantics=("parallel",),
            vmem_limit_bytes=_VMEM_LIMIT),
    )(x, w2q, s2.reshape(1, N2).astype(jnp.float32),
      b2.reshape(1, N2).astype(jnp.float32),
      w3.astype(cdt), b3.reshape(1, N3).astype(jnp.float32))


def kernel(emb, conv1_w, conv1_b, conv2_w, conv2_b, conv3_w, conv3_b,
           fc1_wq, fc1_s, fc1_b, fc2_wq, fc2_s, fc2_b, fc3_w, fc3_b,
           token_ids):
    cdt = jnp.float32
    B = token_ids.shape[0]

    x = _embed_conv1(token_ids, emb, conv1_w, conv1_b, cdt)

    a2 = _banded(conv2_w, 3, 3, 16, 32, 49, 47, cdt)
    x = _conv_lane(x, a2, conv2_b, kh=3, pin=49, ho=47, hp=23, wp=23,
                   cout=32, wo=47, out_dtype=cdt)

    a3 = _banded(conv3_w, 4, 4, 32, 64, 23, 20, cdt)
    x = _conv_lane(x, a3, conv3_b, kh=4, pin=24, ho=20, hp=10, wp=10,
                   cout=64, wo=20, out_dtype=cdt)

    x = x.reshape(B, 16, 640)[:, :10, :].reshape(B, 6400)
    x = _fc1(x, fc1_wq, fc1_s, fc1_b, cdt, tm=128)
    return _fc23(x, fc2_wq, fc2_s, fc2_b, fc3_w, fc3_b, cdt, tm=128)


# speedup vs baseline: 4.1063x; 1.0117x over previous
import functools

import jax
import jax.numpy as jnp
from jax.experimental import pallas as pl
from jax.experimental.pallas import tpu as pltpu

_VMEM_LIMIT = 32 * 1024 * 1024
_NB = 8


def _ceil8(x):
    return ((x + 7) // 8) * 8


def _s1_body(ia_ref, ib_ref, ic_ref, ga_ref, gb_ref, bt_ref, o_ref, *,
             vocab, nb, hp):
    m = nb * hp
    iota = jax.lax.broadcasted_iota(jnp.int32, (m, 2 * vocab), 1)
    ia = ia_ref[...].reshape(m, 1)
    ib = ib_ref[...].reshape(m, 1)
    ic = ic_ref[...].reshape(m, 1)
    f = ga_ref.dtype
    oab = ((iota == ia) | (iota == ib + vocab)).astype(f)
    obc = ((iota == ib) | (iota == ic + vocab)).astype(f)
    ga, gb = ga_ref[...], gb_ref[...]
    o_ref[...] = jnp.dot(oab, ga, preferred_element_type=jnp.float32
                         ).astype(o_ref.dtype)
    for oh, g in ((obc, ga), (oab, gb), (obc, gb)):
        o_ref[...] = jnp.maximum(
            o_ref[...],
            jnp.dot(oh, g, preferred_element_type=jnp.float32
                    ).astype(o_ref.dtype))
    o_ref[...] = jnp.maximum(o_ref[...] + bt_ref[...].astype(o_ref.dtype), 0.0)


def _embed_conv1(token_ids, table, w1, b1, cdt):
    B, S = token_ids.shape
    V, E = table.shape
    C1 = w1.shape[-1]
    Hp, Wp = (S - 1) // 2, (E - 1) // 2

    even = 2 * jnp.arange(Wp)
    col = [jnp.take(table, even + d, axis=1) for d in range(3)]

    def filt(dy, dj):
        g = (col[dj][:, :, None] * w1[dy, 0, :]
             + col[dj + 1][:, :, None] * w1[dy, 1, :])
        return g.reshape(V, Wp * C1)

    ga = jnp.concatenate([filt(0, 0), filt(1, 0)], axis=0).astype(cdt)
    gb = jnp.concatenate([filt(0, 1), filt(1, 1)], axis=0).astype(cdt)
    bt = jnp.tile(b1.astype(jnp.float32), Wp).reshape(1, Wp * C1)

    ids = token_ids.astype(jnp.int32)
    ia = ids[:, 0:2 * Hp:2].reshape(B, Hp, 1)
    ib = ids[:, 1:2 * Hp + 1:2].reshape(B, Hp, 1)
    ic = ids[:, 2:2 * Hp + 2:2].reshape(B, Hp, 1)

    body = functools.partial(_s1_body, vocab=V, nb=_NB, hp=Hp)
    return pl.pallas_call(
        body,
        out_shape=jax.ShapeDtypeStruct((B * Hp, Wp * C1), cdt),
        grid=(B // _NB,),
        in_specs=[
            pl.BlockSpec((_NB, Hp, 1), lambda b: (b, 0, 0)),
            pl.BlockSpec((_NB, Hp, 1), lambda b: (b, 0, 0)),
            pl.BlockSpec((_NB, Hp, 1), lambda b: (b, 0, 0)),
            pl.BlockSpec((2 * V, Wp * C1), lambda b: (0, 0)),
            pl.BlockSpec((2 * V, Wp * C1), lambda b: (0, 0)),
            pl.BlockSpec((1, Wp * C1), lambda b: (0, 0)),
        ],
        out_specs=pl.BlockSpec((_NB * Hp, Wp * C1), lambda b: (b, 0)),
        compiler_params=pltpu.CompilerParams(
            dimension_semantics=("parallel",),
            vmem_limit_bytes=_VMEM_LIMIT),
    )(ia, ib, ic, ga, gb, bt)


def _banded(wmat, kh, kw, cin, cout, win, wo, dtype):
    w4 = wmat.reshape(kh, kw, cin, cout).astype(jnp.float32)
    x = jnp.arange(wo)
    order = jnp.concatenate([x[0::2], x[1::2]])
    j = jnp.arange(win)
    a = jnp.zeros((kh, win, cin, wo, cout), jnp.float32)
    for dx in range(kw):
        e = (j[:, None] == order[None, :] + dx).astype(jnp.float32)
        a = a + jnp.einsum('jx,kic->kjixc', e, w4[:, dx])
    return a.reshape(kh, win * cin, wo * cout).astype(dtype)


def _conv_body(x_ref, a_ref, bt_ref, o_ref, d_ref, hs_ref, hm_ref, *,
               nb, kh, pin, pout, ho, hp, wp, cout, half):
    mv = nb * pin - (kh - 1)
    d_ref[pl.ds(0, mv), :] = jnp.dot(
        x_ref[pl.ds(0, mv), :], a_ref[0],
        preferred_element_type=jnp.float32)
    for dy in range(1, kh):
        d_ref[pl.ds(0, mv), :] = d_ref[pl.ds(0, mv), :] + jnp.dot(
            x_ref[pl.ds(dy, mv), :], a_ref[dy],
            preferred_element_type=jnp.float32)
    nw = wp * cout
    hs_ref[pl.ds(0, mv), :] = jnp.maximum(
        d_ref[pl.ds(0, mv), pl.ds(0, nw)],
        d_ref[pl.ds(0, mv), pl.ds(half, nw)])
    chunks = (nw + 127) // 128
    for c in range(chunks):
        wc = min(128, nw - c * 128)
        hm_ref[c, pl.ds(0, mv - 1), pl.ds(0, wc)] = jnp.maximum(
            hs_ref[pl.ds(0, mv - 1), pl.ds(c * 128, wc)],
            hs_ref[pl.ds(1, mv - 1), pl.ds(c * 128, wc)])
    o_ref[...] = jnp.zeros_like(o_ref)
    for i in range(nb):
        for c in range(chunks):
            wc = min(128, nw - c * 128)
            o_ref[pl.ds(i * pout, hp), pl.ds(c * 128, wc)] = jnp.maximum(
                hm_ref[c, pl.ds(i * pin, hp, 2), pl.ds(0, wc)]
                + bt_ref[0, pl.ds(c * 128, wc)], 0.0).astype(o_ref.dtype)


def _conv_lane(x, a, b, *, kh, pin, ho, hp, wp, cout, wo, out_dtype):
    rows, K = x.shape
    B = rows // pin
    N = a.shape[-1]
    pout = _ceil8(hp)
    half = ((wo + 1) // 2) * cout
    bt = jnp.tile(b.astype(jnp.float32), wp).reshape(1, wp * cout)
    mv_pad = _ceil8(_NB * pin)

    body = functools.partial(
        _conv_body, nb=_NB, kh=kh, pin=pin, pout=pout, ho=ho, hp=hp, wp=wp,
        cout=cout, half=half)
    return pl.pallas_call(
        body,
        out_shape=jax.ShapeDtypeStruct((B * pout, wp * cout), out_dtype),
        grid=(B // _NB,),
        in_specs=[
            pl.BlockSpec((_NB * pin, K), lambda bb: (bb, 0)),
            pl.BlockSpec((kh, K, N), lambda bb: (0, 0, 0)),
            pl.BlockSpec((1, wp * cout), lambda bb: (0, 0)),
        ],
        out_specs=pl.BlockSpec((_NB * pout, wp * cout), lambda bb: (bb, 0)),
        scratch_shapes=[
            pltpu.VMEM((mv_pad, N), jnp.float32),
            pltpu.VMEM((mv_pad, wp * cout), jnp.float32),
            pltpu.VMEM(((wp * cout + 127) // 128, mv_pad, 128), jnp.float32),
        ],
        compiler_params=pltpu.CompilerParams(
            dimension_semantics=("parallel",),
            vmem_limit_bytes=_VMEM_LIMIT),
    )(x, a, bt)


def _fc1_body(x_ref, w_ref, s_ref, b_ref, o_ref):
    w = w_ref[...].astype(x_ref.dtype)
    acc = jnp.dot(x_ref[...], w, preferred_element_type=jnp.float32)
    o_ref[...] = jnp.maximum(acc * s_ref[...] + b_ref[...], 0.0
                             ).astype(o_ref.dtype)


def _fc23_body(x_ref, w2_ref, s2_ref, b2_ref, w3_ref, b3_ref, o_ref, h_ref):
    w2 = w2_ref[...].astype(x_ref.dtype)
    a = jnp.dot(x_ref[...], w2, preferred_element_type=jnp.float32)
    h_ref[...] = jnp.maximum(a * s2_ref[...] + b2_ref[...], 0.0
                             ).astype(h_ref.dtype)
    o_ref[...] = (jnp.dot(h_ref[...], w3_ref[...],
                          preferred_element_type=jnp.float32) + b3_ref[...])


def _fc1(x, wq, s, b, cdt, tm):
    M, K = x.shape
    N = wq.shape[1]
    tn = 128
    return pl.pallas_call(
        _fc1_body,
        out_shape=jax.ShapeDtypeStruct((M, N), cdt),
        grid=(M // tm, N // tn),
        in_specs=[
            pl.BlockSpec((tm, K), lambda i, j: (i, 0)),
            pl.BlockSpec((K, tn), lambda i, j: (0, j)),
            pl.BlockSpec((1, tn), lambda i, j: (0, j)),
            pl.BlockSpec((1, tn), lambda i, j: (0, j)),
        ],
        out_specs=pl.BlockSpec((tm, tn), lambda i, j: (i, j)),
        compiler_params=pltpu.CompilerParams(
            dimension_semantics=("parallel", "parallel"),
            vmem_limit_bytes=_VMEM_LIMIT),
    )(x, wq, s.reshape(1, N).astype(jnp.float32),
      b.reshape(1, N).astype(jnp.float32))


def _fc23(x, w2q, s2, b2, w3, b3, cdt, tm):
    M, K = x.shape
    N2 = w2q.shape[1]
    N3 = w3.shape[1]
    return pl.pallas_call(
        _fc23_body,
        out_shape=jax.ShapeDtypeStruct((M, N3), jnp.float32),
        grid=(M // tm,),
        in_specs=[
            pl.BlockSpec((tm, K), lambda i: (i, 0)),
            pl.BlockSpec((K, N2), lambda i: (0, 0)),
            pl.BlockSpec((1, N2), lambda i: (0, 0)),
            pl.BlockSpec((1, N2), lambda i: (0, 0)),
            pl.BlockSpec((N2, N3), lambda i: (0, 0)),
            pl.BlockSpec((1, N3), lambda i: (0, 0)),
        ],
        out_specs=pl.BlockSpec((tm, N3), lambda i: (i, 0)),
        scratch_shapes=[pltpu.VMEM((tm, N2), cdt)],
        compiler_params=pltpu.CompilerParams(
            dimension_semantics=("parallel",),
            vmem_limit_bytes=_VMEM_LIMIT),
    )(x, w2q, s2.reshape(1, N2).astype(jnp.float32),
      b2.reshape(1, N2).astype(jnp.float32),
      w3.astype(cdt), b3.reshape(1, N3).astype(jnp.float32))


def kernel(emb, conv1_w, conv1_b, conv2_w, conv2_b, conv3_w, conv3_b,
           fc1_wq, fc1_s, fc1_b, fc2_wq, fc2_s, fc2_b, fc3_w, fc3_b,
           token_ids):
    cdt = jnp.bfloat16
    B = token_ids.shape[0]

    x = _embed_conv1(token_ids, emb, conv1_w, conv1_b, cdt)

    a2 = _banded(conv2_w, 3, 3, 16, 32, 49, 47, cdt)
    x = _conv_lane(x, a2, conv2_b, kh=3, pin=49, ho=47, hp=23, wp=23,
                   cout=32, wo=47, out_dtype=cdt)

    a3 = _banded(conv3_w, 4, 4, 32, 64, 23, 20, cdt)
    x = _conv_lane(x, a3, conv3_b, kh=4, pin=24, ho=20, hp=10, wp=10,
                   cout=64, wo=20, out_dtype=cdt)

    x = x.reshape(B, 16, 640)[:, :10, :].reshape(B, 6400)
    x = _fc1(x, fc1_wq, fc1_s, fc1_b, cdt, tm=128)
    return _fc23(x, fc2_wq, fc2_s, fc2_b, fc3_w, fc3_b, cdt, tm=128)


# speedup vs baseline: 4.5251x; 1.1020x over previous
import functools

import jax
import jax.numpy as jnp
from jax.experimental import pallas as pl
from jax.experimental.pallas import tpu as pltpu

_VMEM_LIMIT = 32 * 1024 * 1024
_NB = 8


def _ceil8(x):
    return ((x + 7) // 8) * 8


def _s1_body(ia_ref, ib_ref, ic_ref, ga_ref, gb_ref, bt_ref, o_ref, *,
             vocab, nb, hp):
    m = nb * hp
    iota = jax.lax.broadcasted_iota(jnp.int32, (m, 2 * vocab), 1)
    ia, ib, ic = ia_ref[...], ib_ref[...], ic_ref[...]
    f = ga_ref.dtype
    oab = ((iota == ia) | (iota == ib + vocab)).astype(f)
    obc = ((iota == ib) | (iota == ic + vocab)).astype(f)
    ga, gb = ga_ref[...], gb_ref[...]
    o_ref[...] = jnp.dot(oab, ga, preferred_element_type=jnp.float32
                         ).astype(o_ref.dtype)
    for oh, g in ((obc, ga), (oab, gb), (obc, gb)):
        o_ref[...] = jnp.maximum(
            o_ref[...],
            jnp.dot(oh, g, preferred_element_type=jnp.float32
                    ).astype(o_ref.dtype))
    o_ref[...] = jnp.maximum(o_ref[...] + bt_ref[...].astype(o_ref.dtype), 0.0)


def _embed_conv1(token_ids, table, w1, b1, cdt):
    B, S = token_ids.shape
    V, E = table.shape
    C1 = w1.shape[-1]
    Hp, Wp = (S - 1) // 2, (E - 1) // 2

    even = 2 * jnp.arange(Wp)
    col = [jnp.take(table, even + d, axis=1) for d in range(3)]

    def filt(dy, dj):
        g = (col[dj][:, :, None] * w1[dy, 0, :]
             + col[dj + 1][:, :, None] * w1[dy, 1, :])
        return g.reshape(V, Wp * C1)

    ga = jnp.concatenate([filt(0, 0), filt(1, 0)], axis=0).astype(cdt)
    gb = jnp.concatenate([filt(0, 1), filt(1, 1)], axis=0).astype(cdt)
    bt = jnp.tile(b1.astype(jnp.float32), Wp).reshape(1, Wp * C1)

    ids = token_ids.astype(jnp.int32)
    ia = ids[:, 0:2 * Hp:2].reshape(B * Hp, 1)
    ib = ids[:, 1:2 * Hp + 1:2].reshape(B * Hp, 1)
    ic = ids[:, 2:2 * Hp + 2:2].reshape(B * Hp, 1)

    body = functools.partial(_s1_body, vocab=V, nb=_NB, hp=Hp)
    return pl.pallas_call(
        body,
        out_shape=jax.ShapeDtypeStruct((B * Hp, Wp * C1), cdt),
        grid=(B // _NB,),
        in_specs=[
            pl.BlockSpec((_NB * Hp, 1), lambda b: (b, 0)),
            pl.BlockSpec((_NB * Hp, 1), lambda b: (b, 0)),
            pl.BlockSpec((_NB * Hp, 1), lambda b: (b, 0)),
            pl.BlockSpec((2 * V, Wp * C1), lambda b: (0, 0)),
            pl.BlockSpec((2 * V, Wp * C1), lambda b: (0, 0)),
            pl.BlockSpec((1, Wp * C1), lambda b: (0, 0)),
        ],
        out_specs=pl.BlockSpec((_NB * Hp, Wp * C1), lambda b: (b, 0)),
        compiler_params=pltpu.CompilerParams(
            dimension_semantics=("parallel",),
            vmem_limit_bytes=_VMEM_LIMIT),
    )(ia, ib, ic, ga, gb, bt)


def _banded(wmat, kh, kw, cin, cout, win, wo, dtype):
    w4 = wmat.reshape(kh, kw, cin, cout).astype(jnp.float32)
    x = jnp.arange(wo)
    order = jnp.concatenate([x[0::2], x[1::2]])
    j = jnp.arange(win)
    a = jnp.zeros((kh, win, cin, wo, cout), jnp.float32)
    for dx in range(kw):
        e = (j[:, None] == order[None, :] + dx).astype(jnp.float32)
        a = a + jnp.einsum('jx,kic->kjixc', e, w4[:, dx])
    return a.reshape(kh, win * cin, wo * cout).astype(dtype)


def _conv_body(x_ref, a_ref, bt_ref, o_ref, d_ref, hs_ref, hm_ref, *,
               nb, kh, pin, pout, ho, hp, wp, cout, half):
    mv = nb * pin - (kh - 1)
    d_ref[pl.ds(0, mv), :] = jnp.dot(
        x_ref[pl.ds(0, mv), :], a_ref[0],
        preferred_element_type=jnp.float32)
    for dy in range(1, kh):
        d_ref[pl.ds(0, mv), :] = d_ref[pl.ds(0, mv), :] + jnp.dot(
            x_ref[pl.ds(dy, mv), :], a_ref[dy],
            preferred_element_type=jnp.float32)
    nw = wp * cout
    hs_ref[pl.ds(0, mv), :] = jnp.maximum(
        d_ref[pl.ds(0, mv), pl.ds(0, nw)],
        d_ref[pl.ds(0, mv), pl.ds(half, nw)])
    chunks = (nw + 127) // 128
    for c in range(chunks):
        wc = min(128, nw - c * 128)
        hm_ref[c, pl.ds(0, mv - 1), pl.ds(0, wc)] = jnp.maximum(
            hs_ref[pl.ds(0, mv - 1), pl.ds(c * 128, wc)],
            hs_ref[pl.ds(1, mv - 1), pl.ds(c * 128, wc)])
    o_ref[...] = jnp.zeros_like(o_ref)
    for i in range(nb):
        for c in range(chunks):
            wc = min(128, nw - c * 128)
            o_ref[pl.ds(i * pout, hp), pl.ds(c * 128, wc)] = jnp.maximum(
                hm_ref[c, pl.ds(i * pin, hp, 2), pl.ds(0, wc)]
                + bt_ref[0, pl.ds(c * 128, wc)], 0.0).astype(o_ref.dtype)


def _conv_lane(x, a, b, *, kh, pin, ho, hp, wp, cout, wo, out_dtype):
    rows, K = x.shape
    B = rows // pin
    N = a.shape[-1]
    pout = _ceil8(hp)
    half = ((wo + 1) // 2) * cout
    bt = jnp.tile(b.astype(jnp.float32), wp).reshape(1, wp * cout)
    mv_pad = _ceil8(_NB * pin)

    body = functools.partial(
        _conv_body, nb=_NB, kh=kh, pin=pin, pout=pout, ho=ho, hp=hp, wp=wp,
        cout=cout, half=half)
    return pl.pallas_call(
        body,
        out_shape=jax.ShapeDtypeStruct((B * pout, wp * cout), out_dtype),
        grid=(B // _NB,),
        in_specs=[
            pl.BlockSpec((_NB * pin, K), lambda bb: (bb, 0)),
            pl.BlockSpec((kh, K, N), lambda bb: (0, 0, 0)),
            pl.BlockSpec((1, wp * cout), lambda bb: (0, 0)),
        ],
        out_specs=pl.BlockSpec((_NB * pout, wp * cout), lambda bb: (bb, 0)),
        scratch_shapes=[
            pltpu.VMEM((mv_pad, N), jnp.float32),
            pltpu.VMEM((mv_pad, wp * cout), jnp.float32),
            pltpu.VMEM(((wp * cout + 127) // 128, mv_pad, 128), jnp.float32),
        ],
        compiler_params=pltpu.CompilerParams(
            dimension_semantics=("parallel",),
            vmem_limit_bytes=_VMEM_LIMIT),
    )(x, a, bt)


def _fc1_body(x_ref, w_ref, s_ref, b_ref, o_ref):
    w = w_ref[...].astype(x_ref.dtype)
    acc = jnp.dot(x_ref[...], w, preferred_element_type=jnp.float32)
    o_ref[...] = jnp.maximum(acc * s_ref[...] + b_ref[...], 0.0
                             ).astype(o_ref.dtype)


def _fc23_body(x_ref, w2_ref, s2_ref, b2_ref, w3_ref, b3_ref, o_ref, h_ref):
    w2 = w2_ref[...].astype(x_ref.dtype)
    a = jnp.dot(x_ref[...], w2, preferred_element_type=jnp.float32)
    h_ref[...] = jnp.maximum(a * s2_ref[...] + b2_ref[...], 0.0
                             ).astype(h_ref.dtype)
    o_ref[...] = (jnp.dot(h_ref[...], w3_ref[...],
                          preferred_element_type=jnp.float32) + b3_ref[...])


def _fc1(x, wq, s, b, cdt, tm):
    M, K = x.shape
    N = wq.shape[1]
    tn = 128
    return pl.pallas_call(
        _fc1_body,
        out_shape=jax.ShapeDtypeStruct((M, N), cdt),
        grid=(M // tm, N // tn),
        in_specs=[
            pl.BlockSpec((tm, K), lambda i, j: (i, 0)),
            pl.BlockSpec((K, tn), lambda i, j: (0, j)),
            pl.BlockSpec((1, tn), lambda i, j: (0, j)),
            pl.BlockSpec((1, tn), lambda i, j: (0, j)),
        ],
        out_specs=pl.BlockSpec((tm, tn), lambda i, j: (i, j)),
        compiler_params=pltpu.CompilerParams(
            dimension_semantics=("parallel", "parallel"),
            vmem_limit_bytes=_VMEM_LIMIT),
    )(x, wq, s.reshape(1, N).astype(jnp.float32),
      b.reshape(1, N).astype(jnp.float32))


def _fc23(x, w2q, s2, b2, w3, b3, cdt, tm):
    M, K = x.shape
    N2 = w2q.shape[1]
    N3 = w3.shape[1]
    return pl.pallas_call(
        _fc23_body,
        out_shape=jax.ShapeDtypeStruct((M, N3), jnp.float32),
        grid=(M // tm,),
        in_specs=[
            pl.BlockSpec((tm, K), lambda i: (i, 0)),
            pl.BlockSpec((K, N2), lambda i: (0, 0)),
            pl.BlockSpec((1, N2), lambda i: (0, 0)),
            pl.BlockSpec((1, N2), lambda i: (0, 0)),
            pl.BlockSpec((N2, N3), lambda i: (0, 0)),
            pl.BlockSpec((1, N3), lambda i: (0, 0)),
        ],
        out_specs=pl.BlockSpec((tm, N3), lambda i: (i, 0)),
        scratch_shapes=[pltpu.VMEM((tm, N2), cdt)],
        compiler_params=pltpu.CompilerParams(
            dimension_semantics=("parallel",),
            vmem_limit_bytes=_VMEM_LIMIT),
    )(x, w2q, s2.reshape(1, N2).astype(jnp.float32),
      b2.reshape(1, N2).astype(jnp.float32),
      w3.astype(cdt), b3.reshape(1, N3).astype(jnp.float32))


def kernel(emb, conv1_w, conv1_b, conv2_w, conv2_b, conv3_w, conv3_b,
           fc1_wq, fc1_s, fc1_b, fc2_wq, fc2_s, fc2_b, fc3_w, fc3_b,
           token_ids):
    cdt = jnp.float32
    B = token_ids.shape[0]

    x = _embed_conv1(token_ids, emb, conv1_w, conv1_b, cdt)

    a2 = _banded(conv2_w, 3, 3, 16, 32, 49, 47, cdt)
    x = _conv_lane(x, a2, conv2_b, kh=3, pin=49, ho=47, hp=23, wp=23,
                   cout=32, wo=47, out_dtype=cdt)

    a3 = _banded(conv3_w, 4, 4, 32, 64, 23, 20, cdt)
    x = _conv_lane(x, a3, conv3_b, kh=4, pin=24, ho=20, hp=10, wp=10,
                   cout=64, wo=20, out_dtype=cdt)

    x = x.reshape(B, 16, 640)[:, :10, :].reshape(B, 6400)
    x = _fc1(x, fc1_wq, fc1_s, fc1_b, cdt, tm=128)
    return _fc23(x, fc2_wq, fc2_s, fc2_b, fc3_w, fc3_b, cdt, tm=128)


# speedup vs baseline: 4.5896x; 1.0143x over previous
import functools

import jax
import jax.numpy as jnp
from jax.experimental import pallas as pl
from jax.experimental.pallas import tpu as pltpu

_VMEM_LIMIT = 32 * 1024 * 1024
_NB = 8


def _ceil8(x):
    return ((x + 7) // 8) * 8


def _s1_body(ia_ref, ib_ref, ic_ref, ga_ref, gb_ref, bt_ref, o_ref, *,
             vocab, nb, hp):
    m = nb * hp
    iota = jax.lax.broadcasted_iota(jnp.int32, (m, 2 * vocab), 1)
    ia, ib, ic = ia_ref[...], ib_ref[...], ic_ref[...]
    f = ga_ref.dtype
    oab = ((iota == ia) | (iota == ib + vocab)).astype(f)
    obc = ((iota == ib) | (iota == ic + vocab)).astype(f)
    ga, gb = ga_ref[...], gb_ref[...]
    o_ref[...] = jnp.dot(oab, ga, preferred_element_type=jnp.float32
                         ).astype(o_ref.dtype)
    for oh, g in ((obc, ga), (oab, gb), (obc, gb)):
        o_ref[...] = jnp.maximum(
            o_ref[...],
            jnp.dot(oh, g, preferred_element_type=jnp.float32
                    ).astype(o_ref.dtype))
    o_ref[...] = jnp.maximum(o_ref[...] + bt_ref[...].astype(o_ref.dtype), 0.0)


def _embed_conv1(token_ids, table, w1, b1, cdt):
    B, S = token_ids.shape
    V, E = table.shape
    C1 = w1.shape[-1]
    Hp, Wp = (S - 1) // 2, (E - 1) // 2

    even = 2 * jnp.arange(Wp)
    col = [jnp.take(table, even + d, axis=1) for d in range(3)]

    def filt(dy, dj):
        g = (col[dj][:, :, None] * w1[dy, 0, :]
             + col[dj + 1][:, :, None] * w1[dy, 1, :])
        return g.reshape(V, Wp * C1)

    ga = jnp.concatenate([filt(0, 0), filt(1, 0)], axis=0).astype(cdt)
    gb = jnp.concatenate([filt(0, 1), filt(1, 1)], axis=0).astype(cdt)
    bt = jnp.tile(b1.astype(jnp.float32), Wp).reshape(1, Wp * C1)

    ids = token_ids.astype(jnp.int32)
    ia = ids[:, 0:2 * Hp:2].reshape(B * Hp, 1)
    ib = ids[:, 1:2 * Hp + 1:2].reshape(B * Hp, 1)
    ic = ids[:, 2:2 * Hp + 2:2].reshape(B * Hp, 1)

    body = functools.partial(_s1_body, vocab=V, nb=_NB, hp=Hp)
    return pl.pallas_call(
        body,
        out_shape=jax.ShapeDtypeStruct((B * Hp, Wp * C1), cdt),
        grid=(B // _NB,),
        in_specs=[
            pl.BlockSpec((_NB * Hp, 1), lambda b: (b, 0)),
            pl.BlockSpec((_NB * Hp, 1), lambda b: (b, 0)),
            pl.BlockSpec((_NB * Hp, 1), lambda b: (b, 0)),
            pl.BlockSpec((2 * V, Wp * C1), lambda b: (0, 0)),
            pl.BlockSpec((2 * V, Wp * C1), lambda b: (0, 0)),
            pl.BlockSpec((1, Wp * C1), lambda b: (0, 0)),
        ],
        out_specs=pl.BlockSpec((_NB * Hp, Wp * C1), lambda b: (b, 0)),
        compiler_params=pltpu.CompilerParams(
            dimension_semantics=("parallel",),
            vmem_limit_bytes=_VMEM_LIMIT),
    )(ia, ib, ic, ga, gb, bt)


def _banded(wmat, kh, kw, cin, cout, win, wo, dtype):
    w4 = wmat.reshape(kh, kw, cin, cout).astype(jnp.float32)
    x = jnp.arange(wo)
    order = jnp.concatenate([x[0::2], x[1::2]])
    j = jnp.arange(win)
    a = jnp.zeros((kh, win, cin, wo, cout), jnp.float32)
    for dx in range(kw):
        e = (j[:, None] == order[None, :] + dx).astype(jnp.float32)
        a = a + jnp.einsum('jx,kic->kjixc', e, w4[:, dx])
    return a.reshape(kh, win * cin, wo * cout).astype(dtype)


def _conv_body(x_ref, a_ref, bt_ref, o_ref, xb_ref, d_ref, hs_ref, hm_ref, *,
               nb, kh, pin, pout, ho, hp, wp, cout, half):
    mv = nb * pin - (kh - 1)
    xb_ref[...] = x_ref[...].astype(xb_ref.dtype)
    d_ref[pl.ds(0, mv), :] = jnp.dot(
        xb_ref[pl.ds(0, mv), :], a_ref[0],
        preferred_element_type=jnp.float32)
    for dy in range(1, kh):
        d_ref[pl.ds(0, mv), :] = d_ref[pl.ds(0, mv), :] + jnp.dot(
            xb_ref[pl.ds(dy, mv), :], a_ref[dy],
            preferred_element_type=jnp.float32)
    nw = wp * cout
    hs_ref[pl.ds(0, mv), :] = jnp.maximum(
        d_ref[pl.ds(0, mv), pl.ds(0, nw)],
        d_ref[pl.ds(0, mv), pl.ds(half, nw)])
    chunks = (nw + 127) // 128
    for c in range(chunks):
        wc = min(128, nw - c * 128)
        hm_ref[c, pl.ds(0, mv - 1), pl.ds(0, wc)] = jnp.maximum(
            hs_ref[pl.ds(0, mv - 1), pl.ds(c * 128, wc)],
            hs_ref[pl.ds(1, mv - 1), pl.ds(c * 128, wc)])
    o_ref[...] = jnp.zeros_like(o_ref)
    for i in range(nb):
        for c in range(chunks):
            wc = min(128, nw - c * 128)
            o_ref[pl.ds(i * pout, hp), pl.ds(c * 128, wc)] = jnp.maximum(
                hm_ref[c, pl.ds(i * pin, hp, 2), pl.ds(0, wc)]
                + bt_ref[0, pl.ds(c * 128, wc)], 0.0).astype(o_ref.dtype)


def _conv_lane(x, a, b, *, kh, pin, ho, hp, wp, cout, wo, out_dtype):
    rows, K = x.shape
    B = rows // pin
    N = a.shape[-1]
    pout = _ceil8(hp)
    half = ((wo + 1) // 2) * cout
    bt = jnp.tile(b.astype(jnp.float32), wp).reshape(1, wp * cout)
    mv_pad = _ceil8(_NB * pin)

    body = functools.partial(
        _conv_body, nb=_NB, kh=kh, pin=pin, pout=pout, ho=ho, hp=hp, wp=wp,
        cout=cout, half=half)
    return pl.pallas_call(
        body,
        out_shape=jax.ShapeDtypeStruct((B * pout, wp * cout), out_dtype),
        grid=(B // _NB,),
        in_specs=[
            pl.BlockSpec((_NB * pin, K), lambda bb: (bb, 0)),
            pl.BlockSpec((kh, K, N), lambda bb: (0, 0, 0)),
            pl.BlockSpec((1, wp * cout), lambda bb: (0, 0)),
        ],
        out_specs=pl.BlockSpec((_NB * pout, wp * cout), lambda bb: (bb, 0)),
        scratch_shapes=[
            pltpu.VMEM((_NB * pin, K), a.dtype),
            pltpu.VMEM((mv_pad, N), jnp.float32),
            pltpu.VMEM((mv_pad, wp * cout), jnp.float32),
            pltpu.VMEM(((wp * cout + 127) // 128, mv_pad, 128), jnp.float32),
        ],
        compiler_params=pltpu.CompilerParams(
            dimension_semantics=("parallel",),
            vmem_limit_bytes=_VMEM_LIMIT),
    )(x, a, bt)


def _fc1_body(x_ref, w_ref, s_ref, b_ref, o_ref):
    w = w_ref[...].astype(x_ref.dtype)
    acc = jnp.dot(x_ref[...], w, preferred_element_type=jnp.float32)
    o_ref[...] = jnp.maximum(acc * s_ref[...] + b_ref[...], 0.0
                             ).astype(o_ref.dtype)


def _fc23_body(x_ref, w2_ref, s2_ref, b2_ref, w3_ref, b3_ref, o_ref, h_ref):
    w2 = w2_ref[...].astype(x_ref.dtype)
    a = jnp.dot(x_ref[...], w2, preferred_element_type=jnp.float32)
    h_ref[...] = jnp.maximum(a * s2_ref[...] + b2_ref[...], 0.0
                             ).astype(h_ref.dtype)
    o_ref[...] = (jnp.dot(h_ref[...], w3_ref[...],
                          preferred_element_type=jnp.float32) + b3_ref[...])


def _fc1(x, wq, s, b, cdt, tm):
    M, K = x.shape
    N = wq.shape[1]
    tn = 128
    return pl.pallas_call(
        _fc1_body,
        out_shape=jax.ShapeDtypeStruct((M, N), cdt),
        grid=(M // tm, N // tn),
        in_specs=[
            pl.BlockSpec((tm, K), lambda i, j: (i, 0)),
            pl.BlockSpec((K, tn), lambda i, j: (0, j)),
            pl.BlockSpec((1, tn), lambda i, j: (0, j)),
            pl.BlockSpec((1, tn), lambda i, j: (0, j)),
        ],
        out_specs=pl.BlockSpec((tm, tn), lambda i, j: (i, j)),
        compiler_params=pltpu.CompilerParams(
            dimension_semantics=("parallel", "parallel"),
            vmem_limit_bytes=_VMEM_LIMIT),
    )(x, wq, s.reshape(1, N).astype(jnp.float32),
      b.reshape(1, N).astype(jnp.float32))


def _fc23(x, w2q, s2, b2, w3, b3, cdt, tm):
    M, K = x.shape
    N2 = w2q.shape[1]
    N3 = w3.shape[1]
    return pl.pallas_call(
        _fc23_body,
        out_shape=jax.ShapeDtypeStruct((M, N3), jnp.float32),
        grid=(M // tm,),
        in_specs=[
            pl.BlockSpec((tm, K), lambda i: (i, 0)),
            pl.BlockSpec((K, N2), lambda i: (0, 0)),
            pl.BlockSpec((1, N2), lambda i: (0, 0)),
            pl.BlockSpec((1, N2), lambda i: (0, 0)),
            pl.BlockSpec((N2, N3), lambda i: (0, 0)),
            pl.BlockSpec((1, N3), lambda i: (0, 0)),
        ],
        out_specs=pl.BlockSpec((tm, N3), lambda i: (i, 0)),
        scratch_shapes=[pltpu.VMEM((tm, N2), cdt)],
        compiler_params=pltpu.CompilerParams(
            dimension_semantics=("parallel",),
            vmem_limit_bytes=_VMEM_LIMIT),
    )(x, w2q, s2.reshape(1, N2).astype(jnp.float32),
      b2.reshape(1, N2).astype(jnp.float32),
      w3.astype(cdt), b3.reshape(1, N3).astype(jnp.float32))


def kernel(emb, conv1_w, conv1_b, conv2_w, conv2_b, conv3_w, conv3_b,
           fc1_wq, fc1_s, fc1_b, fc2_wq, fc2_s, fc2_b, fc3_w, fc3_b,
           token_ids):
    cdt = jnp.float32
    B = token_ids.shape[0]

    x = _embed_conv1(token_ids, emb, conv1_w, conv1_b, cdt)

    a2 = _banded(conv2_w, 3, 3, 16, 32, 49, 47, jnp.bfloat16)
    x = _conv_lane(x, a2, conv2_b, kh=3, pin=49, ho=47, hp=23, wp=23,
                   cout=32, wo=47, out_dtype=cdt)

    a3 = _banded(conv3_w, 4, 4, 32, 64, 23, 20, jnp.bfloat16)
    x = _conv_lane(x, a3, conv3_b, kh=4, pin=24, ho=20, hp=10, wp=10,
                   cout=64, wo=20, out_dtype=cdt)

    x = x.reshape(B, 16, 640)[:, :10, :].reshape(B, 6400)
    x = _fc1(x, fc1_wq, fc1_s, fc1_b, cdt, tm=128)
    return _fc23(x, fc2_wq, fc2_s, fc2_b, fc3_w, fc3_b, cdt, tm=128)


# speedup vs baseline: 4.9685x; 1.0825x over previous
import functools

import jax
import jax.numpy as jnp
from jax.experimental import pallas as pl
from jax.experimental.pallas import tpu as pltpu

_VMEM_LIMIT = 32 * 1024 * 1024
_NB = 16


def _ceil8(x):
    return ((x + 7) // 8) * 8


def _s1_body(ia_ref, ib_ref, ic_ref, ga_ref, gb_ref, bt_ref, o_ref, *,
             vocab, nb, hp):
    m = nb * hp
    iota = jax.lax.broadcasted_iota(jnp.int32, (m, 2 * vocab), 1)
    ia, ib, ic = ia_ref[...], ib_ref[...], ic_ref[...]
    f = ga_ref.dtype
    oab = ((iota == ia) | (iota == ib + vocab)).astype(f)
    obc = ((iota == ib) | (iota == ic + vocab)).astype(f)
    ga, gb = ga_ref[...], gb_ref[...]
    o_ref[...] = jnp.dot(oab, ga, preferred_element_type=jnp.float32
                         ).astype(o_ref.dtype)
    for oh, g in ((obc, ga), (oab, gb), (obc, gb)):
        o_ref[...] = jnp.maximum(
            o_ref[...],
            jnp.dot(oh, g, preferred_element_type=jnp.float32
                    ).astype(o_ref.dtype))
    o_ref[...] = jnp.maximum(o_ref[...] + bt_ref[...].astype(o_ref.dtype), 0.0)


def _embed_conv1(token_ids, table, w1, b1, cdt):
    B, S = token_ids.shape
    V, E = table.shape
    C1 = w1.shape[-1]
    Hp, Wp = (S - 1) // 2, (E - 1) // 2

    even = 2 * jnp.arange(Wp)
    col = [jnp.take(table, even + d, axis=1) for d in range(3)]

    def filt(dy, dj):
        g = (col[dj][:, :, None] * w1[dy, 0, :]
             + col[dj + 1][:, :, None] * w1[dy, 1, :])
        return g.reshape(V, Wp * C1)

    ga = jnp.concatenate([filt(0, 0), filt(1, 0)], axis=0).astype(cdt)
    gb = jnp.concatenate([filt(0, 1), filt(1, 1)], axis=0).astype(cdt)
    bt = jnp.tile(b1.astype(jnp.float32), Wp).reshape(1, Wp * C1)

    ids = token_ids.astype(jnp.int32)
    ia = ids[:, 0:2 * Hp:2].reshape(B * Hp, 1)
    ib = ids[:, 1:2 * Hp + 1:2].reshape(B * Hp, 1)
    ic = ids[:, 2:2 * Hp + 2:2].reshape(B * Hp, 1)

    body = functools.partial(_s1_body, vocab=V, nb=_NB, hp=Hp)
    return pl.pallas_call(
        body,
        out_shape=jax.ShapeDtypeStruct((B * Hp, Wp * C1), cdt),
        grid=(B // _NB,),
        in_specs=[
            pl.BlockSpec((_NB * Hp, 1), lambda b: (b, 0)),
            pl.BlockSpec((_NB * Hp, 1), lambda b: (b, 0)),
            pl.BlockSpec((_NB * Hp, 1), lambda b: (b, 0)),
            pl.BlockSpec((2 * V, Wp * C1), lambda b: (0, 0)),
            pl.BlockSpec((2 * V, Wp * C1), lambda b: (0, 0)),
            pl.BlockSpec((1, Wp * C1), lambda b: (0, 0)),
        ],
        out_specs=pl.BlockSpec((_NB * Hp, Wp * C1), lambda b: (b, 0)),
        compiler_params=pltpu.CompilerParams(
            dimension_semantics=("parallel",),
            vmem_limit_bytes=_VMEM_LIMIT),
    )(ia, ib, ic, ga, gb, bt)


def _banded(wmat, kh, kw, cin, cout, win, wo, dtype):
    w4 = wmat.reshape(kh, kw, cin, cout).astype(jnp.float32)
    x = jnp.arange(wo)
    order = jnp.concatenate([x[0::2], x[1::2]])
    j = jnp.arange(win)
    a = jnp.zeros((kh, win, cin, wo, cout), jnp.float32)
    for dx in range(kw):
        e = (j[:, None] == order[None, :] + dx).astype(jnp.float32)
        a = a + jnp.einsum('jx,kic->kjixc', e, w4[:, dx])
    return a.reshape(kh, win * cin, wo * cout).astype(dtype)


def _conv_body(x_ref, a_ref, bt_ref, o_ref, xb_ref, d_ref, hs_ref, hm_ref, *,
               nb, kh, pin, pout, ho, hp, wp, cout, half):
    mv = nb * pin - (kh - 1)
    xb_ref[...] = x_ref[...].astype(xb_ref.dtype)
    d_ref[pl.ds(0, mv), :] = jnp.dot(
        xb_ref[pl.ds(0, mv), :], a_ref[0],
        preferred_element_type=jnp.float32)
    for dy in range(1, kh):
        d_ref[pl.ds(0, mv), :] = d_ref[pl.ds(0, mv), :] + jnp.dot(
            xb_ref[pl.ds(dy, mv), :], a_ref[dy],
            preferred_element_type=jnp.float32)
    nw = wp * cout
    hs_ref[pl.ds(0, mv), :] = jnp.maximum(
        d_ref[pl.ds(0, mv), pl.ds(0, nw)],
        d_ref[pl.ds(0, mv), pl.ds(half, nw)])
    chunks = (nw + 127) // 128
    for c in range(chunks):
        wc = min(128, nw - c * 128)
        hm_ref[c, pl.ds(0, mv - 1), pl.ds(0, wc)] = jnp.maximum(
            hs_ref[pl.ds(0, mv - 1), pl.ds(c * 128, wc)],
            hs_ref[pl.ds(1, mv - 1), pl.ds(c * 128, wc)])
    o_ref[...] = jnp.zeros_like(o_ref)
    for i in range(nb):
        for c in range(chunks):
            wc = min(128, nw - c * 128)
            o_ref[pl.ds(i * pout, hp), pl.ds(c * 128, wc)] = jnp.maximum(
                hm_ref[c, pl.ds(i * pin, hp, 2), pl.ds(0, wc)]
                + bt_ref[0, pl.ds(c * 128, wc)], 0.0).astype(o_ref.dtype)


def _conv_lane(x, a, b, *, kh, pin, ho, hp, wp, cout, wo, out_dtype):
    rows, K = x.shape
    B = rows // pin
    N = a.shape[-1]
    pout = _ceil8(hp)
    half = ((wo + 1) // 2) * cout
    bt = jnp.tile(b.astype(jnp.float32), wp).reshape(1, wp * cout)
    mv_pad = _ceil8(_NB * pin)

    body = functools.partial(
        _conv_body, nb=_NB, kh=kh, pin=pin, pout=pout, ho=ho, hp=hp, wp=wp,
        cout=cout, half=half)
    return pl.pallas_call(
        body,
        out_shape=jax.ShapeDtypeStruct((B * pout, wp * cout), out_dtype),
        grid=(B // _NB,),
        in_specs=[
            pl.BlockSpec((_NB * pin, K), lambda bb: (bb, 0)),
            pl.BlockSpec((kh, K, N), lambda bb: (0, 0, 0)),
            pl.BlockSpec((1, wp * cout), lambda bb: (0, 0)),
        ],
        out_specs=pl.BlockSpec((_NB * pout, wp * cout), lambda bb: (bb, 0)),
        scratch_shapes=[
            pltpu.VMEM((_NB * pin, K), a.dtype),
            pltpu.VMEM((mv_pad, N), jnp.float32),
            pltpu.VMEM((mv_pad, wp * cout), jnp.float32),
            pltpu.VMEM(((wp * cout + 127) // 128, mv_pad, 128), jnp.float32),
        ],
        compiler_params=pltpu.CompilerParams(
            dimension_semantics=("parallel",),
            vmem_limit_bytes=_VMEM_LIMIT),
    )(x, a, bt)


def _fc1_body(x_ref, w_ref, s_ref, b_ref, o_ref):
    w = w_ref[...].astype(x_ref.dtype)
    acc = jnp.dot(x_ref[...], w, preferred_element_type=jnp.float32)
    o_ref[...] = jnp.maximum(acc * s_ref[...] + b_ref[...], 0.0
                             ).astype(o_ref.dtype)


def _fc23_body(x_ref, w2_ref, s2_ref, b2_ref, w3_ref, b3_ref, o_ref, h_ref):
    w2 = w2_ref[...].astype(x_ref.dtype)
    a = jnp.dot(x_ref[...], w2, preferred_element_type=jnp.float32)
    h_ref[...] = jnp.maximum(a * s2_ref[...] + b2_ref[...], 0.0
                             ).astype(h_ref.dtype)
    o_ref[...] = (jnp.dot(h_ref[...], w3_ref[...],
                          preferred_element_type=jnp.float32) + b3_ref[...])


def _fc1(x, wq, s, b, cdt, tm):
    M, K = x.shape
    N = wq.shape[1]
    tn = 128
    return pl.pallas_call(
        _fc1_body,
        out_shape=jax.ShapeDtypeStruct((M, N), cdt),
        grid=(M // tm, N // tn),
        in_specs=[
            pl.BlockSpec((tm, K), lambda i, j: (i, 0)),
            pl.BlockSpec((K, tn), lambda i, j: (0, j)),
            pl.BlockSpec((1, tn), lambda i, j: (0, j)),
            pl.BlockSpec((1, tn), lambda i, j: (0, j)),
        ],
        out_specs=pl.BlockSpec((tm, tn), lambda i, j: (i, j)),
        compiler_params=pltpu.CompilerParams(
            dimension_semantics=("parallel", "parallel"),
            vmem_limit_bytes=_VMEM_LIMIT),
    )(x, wq, s.reshape(1, N).astype(jnp.float32),
      b.reshape(1, N).astype(jnp.float32))


def _fc23(x, w2q, s2, b2, w3, b3, cdt, tm):
    M, K = x.shape
    N2 = w2q.shape[1]
    N3 = w3.shape[1]
    return pl.pallas_call(
        _fc23_body,
        out_shape=jax.ShapeDtypeStruct((M, N3), jnp.float32),
        grid=(M // tm,),
        in_specs=[
            pl.BlockSpec((tm, K), lambda i: (i, 0)),
            pl.BlockSpec((K, N2), lambda i: (0, 0)),
            pl.BlockSpec((1, N2), lambda i: (0, 0)),
            pl.BlockSpec((1, N2), lambda i: (0, 0)),
            pl.BlockSpec((N2, N3), lambda i: (0, 0)),
            pl.BlockSpec((1, N3), lambda i: (0, 0)),
        ],
        out_specs=pl.BlockSpec((tm, N3), lambda i: (i, 0)),
        scratch_shapes=[pltpu.VMEM((tm, N2), cdt)],
        compiler_params=pltpu.CompilerParams(
            dimension_semantics=("parallel",),
            vmem_limit_bytes=_VMEM_LIMIT),
    )(x, w2q, s2.reshape(1, N2).astype(jnp.float32),
      b2.reshape(1, N2).astype(jnp.float32),
      w3.astype(cdt), b3.reshape(1, N3).astype(jnp.float32))


def kernel(emb, conv1_w, conv1_b, conv2_w, conv2_b, conv3_w, conv3_b,
           fc1_wq, fc1_s, fc1_b, fc2_wq, fc2_s, fc2_b, fc3_w, fc3_b,
           token_ids):
    cdt = jnp.float32
    B = token_ids.shape[0]

    x = _embed_conv1(token_ids, emb, conv1_w, conv1_b, cdt)

    a2 = _banded(conv2_w, 3, 3, 16, 32, 49, 47, jnp.bfloat16)
    x = _conv_lane(x, a2, conv2_b, kh=3, pin=49, ho=47, hp=23, wp=23,
                   cout=32, wo=47, out_dtype=cdt)

    a3 = _banded(conv3_w, 4, 4, 32, 64, 23, 20, jnp.bfloat16)
    x = _conv_lane(x, a3, conv3_b, kh=4, pin=24, ho=20, hp=10, wp=10,
                   cout=64, wo=20, out_dtype=cdt)

    x = x.reshape(B, 16, 640)[:, :10, :].reshape(B, 6400)
    x = _fc1(x, fc1_wq, fc1_s, fc1_b, cdt, tm=128)
    return _fc23(x, fc2_wq, fc2_s, fc2_b, fc3_w, fc3_b, cdt, tm=128)


# speedup vs baseline: 5.0579x; 1.0180x over previous
import functools

import jax
import jax.numpy as jnp
from jax.experimental import pallas as pl
from jax.experimental.pallas import tpu as pltpu

_VMEM_LIMIT = 56 * 1024 * 1024
_NB = 32


def _ceil8(x):
    return ((x + 7) // 8) * 8


def _s1_body(ia_ref, ib_ref, ic_ref, ga_ref, gb_ref, bt_ref, o_ref, *,
             vocab, nb, hp):
    m = nb * hp
    iota = jax.lax.broadcasted_iota(jnp.int32, (m, 2 * vocab), 1)
    ia, ib, ic = ia_ref[...], ib_ref[...], ic_ref[...]
    f = ga_ref.dtype
    oab = ((iota == ia) | (iota == ib + vocab)).astype(f)
    obc = ((iota == ib) | (iota == ic + vocab)).astype(f)
    ga, gb = ga_ref[...], gb_ref[...]
    o_ref[...] = jnp.dot(oab, ga, preferred_element_type=jnp.float32
                         ).astype(o_ref.dtype)
    for oh, g in ((obc, ga), (oab, gb), (obc, gb)):
        o_ref[...] = jnp.maximum(
            o_ref[...],
            jnp.dot(oh, g, preferred_element_type=jnp.float32
                    ).astype(o_ref.dtype))
    o_ref[...] = jnp.maximum(o_ref[...] + bt_ref[...].astype(o_ref.dtype), 0.0)


def _embed_conv1(token_ids, table, w1, b1, cdt):
    B, S = token_ids.shape
    V, E = table.shape
    C1 = w1.shape[-1]
    Hp, Wp = (S - 1) // 2, (E - 1) // 2

    even = 2 * jnp.arange(Wp)
    col = [jnp.take(table, even + d, axis=1) for d in range(3)]

    def filt(dy, dj):
        g = (col[dj][:, :, None] * w1[dy, 0, :]
             + col[dj + 1][:, :, None] * w1[dy, 1, :])
        return g.reshape(V, Wp * C1)

    ga = jnp.concatenate([filt(0, 0), filt(1, 0)], axis=0).astype(cdt)
    gb = jnp.concatenate([filt(0, 1), filt(1, 1)], axis=0).astype(cdt)
    bt = jnp.tile(b1.astype(jnp.float32), Wp).reshape(1, Wp * C1)

    ids = token_ids.astype(jnp.int32)
    ia = ids[:, 0:2 * Hp:2].reshape(B * Hp, 1)
    ib = ids[:, 1:2 * Hp + 1:2].reshape(B * Hp, 1)
    ic = ids[:, 2:2 * Hp + 2:2].reshape(B * Hp, 1)

    body = functools.partial(_s1_body, vocab=V, nb=_NB, hp=Hp)
    return pl.pallas_call(
        body,
        out_shape=jax.ShapeDtypeStruct((B * Hp, Wp * C1), cdt),
        grid=(B // _NB,),
        in_specs=[
            pl.BlockSpec((_NB * Hp, 1), lambda b: (b, 0)),
            pl.BlockSpec((_NB * Hp, 1), lambda b: (b, 0)),
            pl.BlockSpec((_NB * Hp, 1), lambda b: (b, 0)),
            pl.BlockSpec((2 * V, Wp * C1), lambda b: (0, 0)),
            pl.BlockSpec((2 * V, Wp * C1), lambda b: (0, 0)),
            pl.BlockSpec((1, Wp * C1), lambda b: (0, 0)),
        ],
        out_specs=pl.BlockSpec((_NB * Hp, Wp * C1), lambda b: (b, 0)),
        compiler_params=pltpu.CompilerParams(
            dimension_semantics=("parallel",),
            vmem_limit_bytes=_VMEM_LIMIT),
    )(ia, ib, ic, ga, gb, bt)


def _banded(wmat, kh, kw, cin, cout, win, wo, dtype):
    w4 = wmat.reshape(kh, kw, cin, cout).astype(jnp.float32)
    x = jnp.arange(wo)
    order = jnp.concatenate([x[0::2], x[1::2]])
    j = jnp.arange(win)
    a = jnp.zeros((kh, win, cin, wo, cout), jnp.float32)
    for dx in range(kw):
        e = (j[:, None] == order[None, :] + dx).astype(jnp.float32)
        a = a + jnp.einsum('jx,kic->kjixc', e, w4[:, dx])
    return a.reshape(kh, win * cin, wo * cout).astype(dtype)


def _conv_body(x_ref, a_ref, bt_ref, o_ref, xb_ref, d_ref, hs_ref, hm_ref, *,
               nb, kh, pin, pout, ho, hp, wp, cout, half):
    mv = nb * pin - (kh - 1)
    xb_ref[...] = x_ref[...].astype(xb_ref.dtype)
    d_ref[pl.ds(0, mv), :] = jnp.dot(
        xb_ref[pl.ds(0, mv), :], a_ref[0],
        preferred_element_type=jnp.float32)
    for dy in range(1, kh):
        d_ref[pl.ds(0, mv), :] = d_ref[pl.ds(0, mv), :] + jnp.dot(
            xb_ref[pl.ds(dy, mv), :], a_ref[dy],
            preferred_element_type=jnp.float32)
    nw = wp * cout
    hs_ref[pl.ds(0, mv), :] = jnp.maximum(
        d_ref[pl.ds(0, mv), pl.ds(0, nw)],
        d_ref[pl.ds(0, mv), pl.ds(half, nw)])
    chunks = (nw + 127) // 128
    for c in range(chunks):
        wc = min(128, nw - c * 128)
        hm_ref[c, pl.ds(0, mv - 1), pl.ds(0, wc)] = jnp.maximum(
            hs_ref[pl.ds(0, mv - 1), pl.ds(c * 128, wc)],
            hs_ref[pl.ds(1, mv - 1), pl.ds(c * 128, wc)])
    o_ref[...] = jnp.zeros_like(o_ref)
    for i in range(nb):
        for c in range(chunks):
            wc = min(128, nw - c * 128)
            o_ref[pl.ds(i * pout, hp), pl.ds(c * 128, wc)] = jnp.maximum(
                hm_ref[c, pl.ds(i * pin, hp, 2), pl.ds(0, wc)]
                + bt_ref[0, pl.ds(c * 128, wc)], 0.0).astype(o_ref.dtype)


def _conv_lane(x, a, b, *, kh, pin, ho, hp, wp, cout, wo, out_dtype):
    rows, K = x.shape
    B = rows // pin
    N = a.shape[-1]
    pout = _ceil8(hp)
    half = ((wo + 1) // 2) * cout
    bt = jnp.tile(b.astype(jnp.float32), wp).reshape(1, wp * cout)
    mv_pad = _ceil8(_NB * pin)

    body = functools.partial(
        _conv_body, nb=_NB, kh=kh, pin=pin, pout=pout, ho=ho, hp=hp, wp=wp,
        cout=cout, half=half)
    return pl.pallas_call(
        body,
        out_shape=jax.ShapeDtypeStruct((B * pout, wp * cout), out_dtype),
        grid=(B // _NB,),
        in_specs=[
            pl.BlockSpec((_NB * pin, K), lambda bb: (bb, 0)),
            pl.BlockSpec((kh, K, N), lambda bb: (0, 0, 0)),
            pl.BlockSpec((1, wp * cout), lambda bb: (0, 0)),
        ],
        out_specs=pl.BlockSpec((_NB * pout, wp * cout), lambda bb: (bb, 0)),
        scratch_shapes=[
            pltpu.VMEM((_NB * pin, K), a.dtype),
            pltpu.VMEM((mv_pad, N), jnp.float32),
            pltpu.VMEM((mv_pad, wp * cout), jnp.float32),
            pltpu.VMEM(((wp * cout + 127) // 128, mv_pad, 128), jnp.float32),
        ],
        compiler_params=pltpu.CompilerParams(
            dimension_semantics=("parallel",),
            vmem_limit_bytes=_VMEM_LIMIT),
    )(x, a, bt)


def _fc1_body(x_ref, w_ref, s_ref, b_ref, o_ref):
    w = w_ref[...].astype(x_ref.dtype)
    acc = jnp.dot(x_ref[...], w, preferred_element_type=jnp.float32)
    o_ref[...] = jnp.maximum(acc * s_ref[...] + b_ref[...], 0.0
                             ).astype(o_ref.dtype)


def _fc23_body(x_ref, w2_ref, s2_ref, b2_ref, w3_ref, b3_ref, o_ref, h_ref):
    w2 = w2_ref[...].astype(x_ref.dtype)
    a = jnp.dot(x_ref[...], w2, preferred_element_type=jnp.float32)
    h_ref[...] = jnp.maximum(a * s2_ref[...] + b2_ref[...], 0.0
                             ).astype(h_ref.dtype)
    o_ref[...] = (jnp.dot(h_ref[...], w3_ref[...],
                          preferred_element_type=jnp.float32) + b3_ref[...])


def _fc1(x, wq, s, b, cdt, tm):
    M, K = x.shape
    N = wq.shape[1]
    tn = 128
    return pl.pallas_call(
        _fc1_body,
        out_shape=jax.ShapeDtypeStruct((M, N), cdt),
        grid=(M // tm, N // tn),
        in_specs=[
            pl.BlockSpec((tm, K), lambda i, j: (i, 0)),
            pl.BlockSpec((K, tn), lambda i, j: (0, j)),
            pl.BlockSpec((1, tn), lambda i, j: (0, j)),
            pl.BlockSpec((1, tn), lambda i, j: (0, j)),
        ],
        out_specs=pl.BlockSpec((tm, tn), lambda i, j: (i, j)),
        compiler_params=pltpu.CompilerParams(
            dimension_semantics=("parallel", "parallel"),
            vmem_limit_bytes=_VMEM_LIMIT),
    )(x, wq, s.reshape(1, N).astype(jnp.float32),
      b.reshape(1, N).astype(jnp.float32))


def _fc23(x, w2q, s2, b2, w3, b3, cdt, tm):
    M, K = x.shape
    N2 = w2q.shape[1]
    N3 = w3.shape[1]
    return pl.pallas_call(
        _fc23_body,
        out_shape=jax.ShapeDtypeStruct((M, N3), jnp.float32),
        grid=(M // tm,),
        in_specs=[
            pl.BlockSpec((tm, K), lambda i: (i, 0)),
            pl.BlockSpec((K, N2), lambda i: (0, 0)),
            pl.BlockSpec((1, N2), lambda i: (0, 0)),
            pl.BlockSpec((1, N2), lambda i: (0, 0)),
            pl.BlockSpec((N2, N3), lambda i: (0, 0)),
            pl.BlockSpec((1, N3), lambda i: (0, 0)),
        ],
        out_specs=pl.BlockSpec((tm, N3), lambda i: (i, 0)),
        scratch_shapes=[pltpu.VMEM((tm, N2), cdt)],
        compiler_params=pltpu.CompilerParams(
            dimension_semantics=("parallel",),
            vmem_limit_bytes=_VMEM_LIMIT),
    )(x, w2q, s2.reshape(1, N2).astype(jnp.float32),
      b2.reshape(1, N2).astype(jnp.float32),
      w3.astype(cdt), b3.reshape(1, N3).astype(jnp.float32))


def kernel(emb, conv1_w, conv1_b, conv2_w, conv2_b, conv3_w, conv3_b,
           fc1_wq, fc1_s, fc1_b, fc2_wq, fc2_s, fc2_b, fc3_w, fc3_b,
           token_ids):
    cdt = jnp.float32
    B = token_ids.shape[0]

    x = _embed_conv1(token_ids, emb, conv1_w, conv1_b, cdt)

    a2 = _banded(conv2_w, 3, 3, 16, 32, 49, 47, jnp.bfloat16)
    x = _conv_lane(x, a2, conv2_b, kh=3, pin=49, ho=47, hp=23, wp=23,
                   cout=32, wo=47, out_dtype=cdt)

    a3 = _banded(conv3_w, 4, 4, 32, 64, 23, 20, jnp.bfloat16)
    x = _conv_lane(x, a3, conv3_b, kh=4, pin=24, ho=20, hp=10, wp=10,
                   cout=64, wo=20, out_dtype=cdt)

    x = x.reshape(B, 16, 640)[:, :10, :].reshape(B, 6400)
    x = _fc1(x, fc1_wq, fc1_s, fc1_b, cdt, tm=128)
    return _fc23(x, fc2_wq, fc2_s, fc2_b, fc3_w, fc3_b, cdt, tm=128)
